# Initial kernel scaffold; baseline (speedup 1.0000x reference)
import jax
import jax.numpy as jnp
from jax.experimental import pallas as pl


def kernel(x, edge_index, W1l, W1r, b1, W2l, W2r, b2, Wf1, bf1, Wf2, bf2):
    raise NotImplementedError("write your pallas kernel here")


# speedup vs baseline: 3.8157x; 3.8157x over previous
import functools

import jax
import jax.numpy as jnp
from jax import lax
from jax.experimental import pallas as pl
from jax.experimental.pallas import tpu as pltpu
from jax.experimental.pallas import tpu_sc as plsc

N_NODES = 10000
N_EDGES = 320000
D_FEAT = 128
EMB = 128
HIDDEN = 256
ACTIONS = 1000
N_GRAPHS = 10
NODES_PER_GRAPH = 1000

NC = 2
NS = 16
NW = NC * NS

K = 128
ET = -(-N_EDGES // NW)
NCH = -(-ET // K)
ET_P = NCH * K
PAD_E = ET_P * NW
DUMP = N_NODES
ACC_N = 10112
ROWS_PER_TILE = ACC_N // NS


def _sc_segment_sum(with_counts):
    mesh = plsc.VectorSubcoreMesh(core_axis_name="c", subcore_axis_name="s")
    out_type = [jax.ShapeDtypeStruct((NC, ACC_N, D_FEAT), jnp.float32)]
    scratch = [
        pltpu.VMEM((K,), jnp.int32),
        pltpu.VMEM((K,), jnp.int32),
        pltpu.VMEM((K, D_FEAT), jnp.float32),
        pltpu.VMEM_SHARED((ACC_N, D_FEAT), jnp.float32),
        pltpu.SemaphoreType.DMA,
    ]
    if with_counts:
        out_type.append(jax.ShapeDtypeStruct((NC, NS, ACC_N), jnp.float32))
        scratch += [
            pltpu.VMEM((ACC_N,), jnp.float32),
        ]

    @functools.partial(pl.kernel, mesh=mesh, out_type=out_type,
                       scratch_types=scratch,
                       compiler_params=pltpu.CompilerParams(
                           needs_layout_passes=False))
    def body(feats, src, dst, zrows, zcnt, *rest):
        if with_counts:
            out, cnt_out, srcb, dstb, rows, acc, sem, cntl = rest
        else:
            out, srcb, dstb, rows, acc, sem = rest
        c = lax.axis_index("c")
        s = lax.axis_index("s")
        wid = c * NS + s

        rbase = s * ROWS_PER_TILE
        pltpu.sync_copy(zrows.at[pl.ds(rbase, ROWS_PER_TILE)],
                        acc.at[pl.ds(rbase, ROWS_PER_TILE)])
        if with_counts:
            pltpu.sync_copy(zcnt, cntl)
        plsc.subcore_barrier()

        ones16 = jnp.full((16,), 1.0, jnp.float32)
        ebase = wid * ET_P

        def chunk(i, carry):
            base = ebase + i * K
            pltpu.sync_copy(src.at[pl.ds(base, K)], srcb)
            pltpu.sync_copy(dst.at[pl.ds(base, K)], dstb)
            pltpu.async_copy(feats.at[srcb], rows, sem).wait()
            pltpu.sync_copy(rows, acc.at[dstb], add=True)
            if with_counts:
                for j in range(K // 16):
                    dv = dstb[pl.ds(j * 16, 16)]
                    plsc.addupdate_scatter(cntl, [dv], ones16)
            return carry

        lax.fori_loop(0, NCH, chunk, 0)

        if with_counts:
            pltpu.sync_copy(cntl, cnt_out.at[c, s])
        plsc.subcore_barrier()

        pltpu.sync_copy(acc.at[pl.ds(rbase, ROWS_PER_TILE)],
                        out.at[c, pl.ds(rbase, ROWS_PER_TILE)])

    return body


_sc_pass1 = _sc_segment_sum(with_counts=True)
_sc_pass2 = _sc_segment_sum(with_counts=False)


def _conv1_body(p0, p1, cp, x, wl, wr, b, out, m1):
    cnt = jnp.maximum(jnp.sum(cp[...], axis=0), 1.0)
    aggr = (p0[...] + p1[...]) / cnt
    res = (jnp.dot(aggr, wl[...], preferred_element_type=jnp.float32)
           + jnp.dot(x[...], wr[...], preferred_element_type=jnp.float32)
           + b[...][None, :])
    out[...] = res
    m1[...] = jnp.mean(res, axis=0, keepdims=True)[None]


def _mean2_body(p0, p1, cp, m2):
    cnt = jnp.maximum(jnp.sum(cp[...], axis=0), 1.0)
    aggr = (p0[...] + p1[...]) / cnt
    m2[...] = jnp.mean(aggr, axis=0, keepdims=True)[None]


def _head_body(m1, m2, w2l, w2r, b2, wf1, bf1, wf2, bf2, q):
    me = (jnp.dot(m2[...], w2l[...], preferred_element_type=jnp.float32)
          + jnp.dot(m1[...], w2r[...], preferred_element_type=jnp.float32)
          + b2[...][None, :])
    h = jnp.maximum(
        jnp.dot(me, wf1[...], preferred_element_type=jnp.float32)
        + bf1[...][None, :], 0.0)
    q[...] = (jnp.dot(h, wf2[...], preferred_element_type=jnp.float32)
              + bf2[...][None, :])


def kernel(x, edge_index, W1l, W1r, b1, W2l, W2r, b2, Wf1, bf1, Wf2, bf2):
    src = edge_index[0].astype(jnp.int32)
    dst = edge_index[1].astype(jnp.int32)
    pad = PAD_E - N_EDGES
    src_p = jnp.concatenate([src, jnp.zeros((pad,), jnp.int32)])
    dst_p = jnp.concatenate([dst, jnp.full((pad,), DUMP, jnp.int32)])
    zrows = jnp.zeros((ACC_N, D_FEAT), jnp.float32)
    zcnt = jnp.zeros((ACC_N,), jnp.float32)

    p, cnt_p = _sc_pass1(x, src_p, dst_p, zrows, zcnt)
    cnt3 = cnt_p.reshape(NW, ACC_N, 1)

    B = NODES_PER_GRAPH
    row_blk = lambda i: (i, 0)
    cnt_blk = lambda i: (0, i, 0)
    full2 = lambda i: (0, 0)
    conv1, m1 = pl.pallas_call(
        _conv1_body,
        grid=(N_GRAPHS,),
        in_specs=[
            pl.BlockSpec((B, D_FEAT), row_blk),
            pl.BlockSpec((B, D_FEAT), row_blk),
            pl.BlockSpec((NW, B, 1), cnt_blk),
            pl.BlockSpec((B, D_FEAT), row_blk),
            pl.BlockSpec((D_FEAT, EMB), full2),
            pl.BlockSpec((D_FEAT, EMB), full2),
            pl.BlockSpec((EMB,), lambda i: (0,)),
        ],
        out_specs=[
            pl.BlockSpec((B, EMB), row_blk),
            pl.BlockSpec((1, 1, EMB), lambda i: (i, 0, 0)),
        ],
        out_shape=[
            jax.ShapeDtypeStruct((N_NODES, EMB), jnp.float32),
            jax.ShapeDtypeStruct((N_GRAPHS, 1, EMB), jnp.float32),
        ],
    )(p[0], p[1], cnt3, x, W1l, W1r, b1)
    m1 = m1.reshape(N_GRAPHS, EMB)

    q = _sc_pass2(conv1, src_p, dst_p, zrows, zcnt)[0]

    m2 = pl.pallas_call(
        _mean2_body,
        grid=(N_GRAPHS,),
        in_specs=[
            pl.BlockSpec((B, EMB), row_blk),
            pl.BlockSpec((B, EMB), row_blk),
            pl.BlockSpec((NW, B, 1), cnt_blk),
        ],
        out_specs=pl.BlockSpec((1, 1, EMB), lambda i: (i, 0, 0)),
        out_shape=jax.ShapeDtypeStruct((N_GRAPHS, 1, EMB), jnp.float32),
    )(q[0], q[1], cnt3)
    m2 = m2.reshape(N_GRAPHS, EMB)

    q_values = pl.pallas_call(
        _head_body,
        out_shape=jax.ShapeDtypeStruct((N_GRAPHS, ACTIONS), jnp.float32),
    )(m1, m2, W2l, W2r, b2, Wf1, bf1, Wf2, bf2)
    return q_values
